# baseline (speedup 1.0000x reference)
import functools

import jax
import jax.numpy as jnp
from jax import lax
from jax.experimental import pallas as pl
from jax.experimental.pallas import tpu as pltpu
from jax.experimental.pallas import tpu_sc as plsc

VOCAB = 100000
EMBED = 128
SEQ = 22
OUT = 4800
BATCH = 4096

NC = 2
NS = 16
NW = NC * NS
TOTAL_IDX = BATCH * SEQ
IDX_PER_W = TOTAL_IDX // NW
CHUNKS = IDX_PER_W // 128
EMBED_W = EMBED // 2


def _pack_chunk(src, dst):
    lanes = lax.iota(jnp.int32, 16)
    ig = (lanes % 8) * 2
    lo_half = lanes < 8

    def _vgather(v, idx):
        return lax.gather(
            v,
            idx.reshape(16, 1),
            lax.GatherDimensionNumbers(
                offset_dims=(), collapsed_slice_dims=(0,), start_index_map=(0,)
            ),
            (1,),
            mode=lax.GatherScatterMode.PROMISE_IN_BOUNDS,
        )

    def row_body(r2, carry):
        for p in range(2):
            for g in range(4):
                s0 = src[r2 * 2 + p, pl.ds(g * 32, 16)]
                s1 = src[r2 * 2 + p, pl.ds(g * 32 + 16, 16)]
                ev = jnp.where(
                    lo_half,
                    _vgather(s0, ig),
                    _vgather(s1, ig),
                )
                od = jnp.where(
                    lo_half,
                    _vgather(s0, ig + 1),
                    _vgather(s1, ig + 1),
                )
                ua = lax.bitcast_convert_type(ev, jnp.int32)
                ub = lax.bitcast_convert_type(od, jnp.int32)
                lo = lax.shift_right_logical(ua + 0x8000, 16)
                hi = (ub + 0x8000) & jnp.int32(-65536)
                dst[r2, pl.ds(p * 64 + g * 16, 16)] = lo | hi
        return carry

    lax.fori_loop(0, 64, row_body, 0)


def _gather_body(idx_hbm, table_hbm, out_hbm, idx_v, buf_a, buf_b, pk_v, sem_a, sem_b):
    wid = lax.axis_index("s") * NC + lax.axis_index("c")
    base = wid * IDX_PER_W
    base2 = wid * (IDX_PER_W // 2)
    pltpu.sync_copy(idx_hbm.at[wid], idx_v)
    bufs = (buf_a, buf_b)
    sems = (sem_a, sem_b)
    pltpu.make_async_copy(table_hbm.at[idx_v.at[0]], bufs[0], sems[0]).start()
    for j in range(CHUNKS):
        if j + 1 < CHUNKS:
            pltpu.make_async_copy(
                table_hbm.at[idx_v.at[j + 1]], bufs[(j + 1) % 2], sems[(j + 1) % 2]
            ).start()
        pltpu.make_async_copy(
            table_hbm.at[idx_v.at[j]], bufs[j % 2], sems[j % 2]
        ).wait()
        _pack_chunk(bufs[j % 2], pk_v)
        pltpu.sync_copy(pk_v, out_hbm.at[pl.ds(base2 + j * 64, 64)])


@functools.lru_cache(maxsize=None)
def _make_gather():
    return functools.partial(
        pl.kernel,
        mesh=plsc.VectorSubcoreMesh(core_axis_name="c", subcore_axis_name="s"),
        out_type=jax.ShapeDtypeStruct((TOTAL_IDX // 2, EMBED), jnp.int32),
        scratch_types=[
            pltpu.VMEM((CHUNKS, 128), jnp.int32),
            pltpu.VMEM((128, EMBED), jnp.float32),
            pltpu.VMEM((128, EMBED), jnp.float32),
            pltpu.VMEM((64, EMBED), jnp.int32),
            pltpu.SemaphoreType.DMA,
            pltpu.SemaphoreType.DMA,
        ],
    )(_gather_body)


def _mm_body(a_ref, w_ref, b_ref, o_ref):
    acc = lax.dot_general(
        a_ref[...], w_ref[...].astype(jnp.bfloat16),
        dimension_numbers=(((1,), (1,)), ((), ())),
        preferred_element_type=jnp.float32,
    )
    o_ref[...] = jnp.maximum(acc + b_ref[...], 0.0)


def _matmul(flat_bf, W, b2, bn):
    k = flat_bf.shape[1]
    nj = pl.cdiv(OUT, bn)
    return pl.pallas_call(
        _mm_body,
        grid=(nj,),
        in_specs=[
            pl.BlockSpec((BATCH, k), lambda j: (0, 0)),
            pl.BlockSpec((bn, k), lambda j: (j, 0)),
            pl.BlockSpec((1, bn), lambda j: (0, j)),
        ],
        out_specs=pl.BlockSpec((BATCH, bn), lambda j: (0, j)),
        out_shape=jax.ShapeDtypeStruct((BATCH, OUT), jnp.float32),
        compiler_params=pltpu.CompilerParams(vmem_limit_bytes=63 * 1024 * 1024),
    )(flat_bf, W, b2)


def kernel(captions, lengths, table, W, b):
    idx = captions.reshape(NW, CHUNKS, 128).astype(jnp.int32)
    rows = _make_gather()(idx, table)
    flat_bf = lax.bitcast_convert_type(rows, jnp.bfloat16).reshape(
        BATCH, SEQ * EMBED
    )
    out = _matmul(flat_bf, W, b.reshape(1, OUT), bn=512)
    return out.reshape(BATCH, 3, 40, 40)


# speedup vs baseline: 21.6097x; 21.6097x over previous
import functools

import jax
import jax.numpy as jnp
from jax import lax
from jax.experimental import pallas as pl
from jax.experimental.pallas import tpu as pltpu
from jax.experimental.pallas import tpu_sc as plsc

VOCAB = 100000
EMBED = 128
SEQ = 22
OUT = 4800
BATCH = 4096

NC = 2
NS = 16
NW = NC * NS
TOTAL_IDX = BATCH * SEQ
IDX_PER_W = TOTAL_IDX // NW
CHUNKS = IDX_PER_W // 128


def _gather_body(idx_hbm, table_hbm, out_hbm, idx_v, buf_a, buf_b, sem_a, sem_b):
    wid = lax.axis_index("s") * NC + lax.axis_index("c")
    base = wid * IDX_PER_W
    pltpu.sync_copy(idx_hbm.at[wid], idx_v)
    bufs = (buf_a, buf_b)
    sems = (sem_a, sem_b)
    pltpu.make_async_copy(table_hbm.at[idx_v.at[0]], bufs[0], sems[0]).start()
    for j in range(1, CHUNKS + 1):
        if j < CHUNKS:
            pltpu.make_async_copy(
                table_hbm.at[idx_v.at[j]], bufs[j % 2], sems[j % 2]
            ).start()
        prev = j - 1
        pltpu.make_async_copy(
            table_hbm.at[idx_v.at[prev]], bufs[prev % 2], sems[prev % 2]
        ).wait()
        pltpu.sync_copy(
            bufs[prev % 2], out_hbm.at[pl.ds(base + prev * 128, 128)]
        )


@functools.lru_cache(maxsize=None)
def _make_gather():
    return functools.partial(
        pl.kernel,
        mesh=plsc.VectorSubcoreMesh(core_axis_name="c", subcore_axis_name="s"),
        out_type=jax.ShapeDtypeStruct((TOTAL_IDX, EMBED), jnp.float32),
        scratch_types=[
            pltpu.VMEM((CHUNKS, 128), jnp.int32),
            pltpu.VMEM((128, EMBED), jnp.float32),
            pltpu.VMEM((128, EMBED), jnp.float32),
            pltpu.SemaphoreType.DMA,
            pltpu.SemaphoreType.DMA,
        ],
    )(_gather_body)


def _mm_body(a_ref, w_ref, b_ref, o_ref, a_bf):
    j = pl.program_id(1)

    @pl.when(j == 0)
    def _cast_a():
        a_bf[...] = a_ref[...].astype(jnp.bfloat16)

    acc = lax.dot_general(
        a_bf[...], w_ref[...],
        dimension_numbers=(((1,), (1,)), ((), ())),
        preferred_element_type=jnp.float32,
    )
    o_ref[...] = jnp.maximum(acc + b_ref[...], 0.0)


def _matmul(flat, Wbf, b2, bm, bn):
    k = flat.shape[1]
    nj = pl.cdiv(OUT, bn)
    return pl.pallas_call(
        _mm_body,
        grid=(BATCH // bm, nj),
        in_specs=[
            pl.BlockSpec((bm, k), lambda i, j: (i, 0)),
            pl.BlockSpec((bn, k), lambda i, j: (j, 0)),
            pl.BlockSpec((1, bn), lambda i, j: (0, j)),
        ],
        out_specs=pl.BlockSpec((bm, bn), lambda i, j: (i, j)),
        out_shape=jax.ShapeDtypeStruct((BATCH, OUT), jnp.float32),
        scratch_shapes=[
            pltpu.VMEM((bm, k), jnp.bfloat16),
        ],
        compiler_params=pltpu.CompilerParams(vmem_limit_bytes=63 * 1024 * 1024),
    )(flat, Wbf, b2)


def kernel(captions, lengths, table, W, b):
    idx = captions.reshape(NW, CHUNKS, 128).astype(jnp.int32)
    rows = _make_gather()(idx, table)
    flat = rows.reshape(BATCH, SEQ * EMBED)
    out = _matmul(flat, W.astype(jnp.bfloat16), b.reshape(1, OUT), bm=1024, bn=1024)
    return out.reshape(BATCH, 3, 40, 40)
